# baseline (speedup 1.0000x reference)
import jax
import jax.numpy as jnp
from jax import lax
from jax.experimental import pallas as pl
from jax.experimental.pallas import tpu as pltpu
from jax.experimental.pallas import tpu_sc as plsc

_RADIUS = 0.01
_B, _H, _W, _K = 4, 512, 512, 16
_P = 100000
_N = _B * _H * _W

_NC, _NS, _L = 2, 16, 16
_NW = _NC * _NS
_GROUPS = _NW // 2
_PX_PER_GROUP = _N // _GROUPS
_CHUNK = 256
_CHUNKS = _PX_PER_GROUP // _CHUNK


def _sc_body(idxT, distsT, tab01, tab2, out0, out1, out2,
             tab_v, idx_v0, idx_v1, dst_v0, dst_v1, oa_v, ob_v,
             sem_in, sem_out):
    idx_bufs = (idx_v0, idx_v1)
    dst_bufs = (dst_v0, dst_v1)
    wid = lax.axis_index("s") * _NC + lax.axis_index("c")
    role = wid % 2
    group = wid // 2
    gbase = group * _PX_PER_GROUP
    inv_r2 = 1.0 / (_RADIUS * _RADIUS)

    @pl.when(role == 0)
    def _():
        pltpu.sync_copy(tab01, tab_v)

    @pl.when(role == 1)
    def _():
        pltpu.sync_copy(tab2, tab_v)

    def in_descs(ci, buf):
        foff = (gbase + ci * _CHUNK) * _K
        return (
            pltpu.make_async_copy(
                idxT.at[pl.ds(foff, _CHUNK * _K)], idx_bufs[buf],
                sem_in.at[buf]),
            pltpu.make_async_copy(
                distsT.at[pl.ds(foff, _CHUNK * _K)], dst_bufs[buf],
                sem_in.at[buf]),
        )

    def start_in(ci, buf):
        for d in in_descs(ci, buf):
            d.start()

    def wait_in(ci, buf):
        for d in in_descs(ci, buf):
            d.wait()

    def out_descs(ci, buf):
        off = gbase + ci * _CHUNK
        return (
            pltpu.make_async_copy(
                oa_v.at[buf], out0.at[pl.ds(off, _CHUNK)], sem_out.at[buf]),
            pltpu.make_async_copy(
                ob_v.at[buf], out1.at[pl.ds(off, _CHUNK)], sem_out.at[buf]),
            pltpu.make_async_copy(
                oa_v.at[buf], out2.at[pl.ds(off, _CHUNK)], sem_out.at[buf]),
        )

    def wait_out(ci, buf):
        d0, d1, d2 = out_descs(ci, buf)

        @pl.when(role == 0)
        def _():
            d0.wait()
            d1.wait()

        @pl.when(role == 1)
        def _():
            d2.wait()

    def compute(ci, buf):
        off = gbase + ci * _CHUNK
        ib = idx_bufs[buf]
        db = dst_bufs[buf]
        d0, d1, d2 = out_descs(ci, buf)

        @pl.when(role == 0)
        def _():
            @plsc.parallel_loop(0, _CHUNK // _L)
            def _(j):
                jb = j * _L
                fbase = (jb + lax.iota(jnp.int32, _L)) * _K
                zero = jnp.zeros((_L,), jnp.float32)
                acc0, acc1, den = zero, zero, zero
                for k in range(_K):
                    fk = fbase + k
                    iv = plsc.load_gather(ib, [fk])
                    dv = plsc.load_gather(db, [fk])
                    w = 1.0 - dv * inv_r2
                    word = plsc.load_gather(tab_v, [iv])
                    c0 = plsc.bitcast(word & jnp.int32(-65536), jnp.float32)
                    c1 = plsc.bitcast(lax.shift_left(word, 16), jnp.float32)
                    acc0 = acc0 + w * c0
                    acc1 = acc1 + w * c1
                    den = den + w
                rden = 1.0 / jnp.maximum(den, 1e-10)
                oa_v[buf, pl.ds(jb, _L)] = acc0 * rden
                ob_v[buf, pl.ds(jb, _L)] = acc1 * rden

            d0.start()
            d1.start()

        @pl.when(role == 1)
        def _():
            @plsc.parallel_loop(0, _CHUNK // _L)
            def _(j):
                jb = j * _L
                fbase = (jb + lax.iota(jnp.int32, _L)) * _K
                zero = jnp.zeros((_L,), jnp.float32)
                acc0, den = zero, zero
                for k in range(_K):
                    fk = fbase + k
                    iv = plsc.load_gather(ib, [fk])
                    dv = plsc.load_gather(db, [fk])
                    w = 1.0 - dv * inv_r2
                    word = plsc.load_gather(tab_v, [iv])
                    acc0 = acc0 + w * plsc.bitcast(word, jnp.float32)
                    den = den + w
                rden = 1.0 / jnp.maximum(den, 1e-10)
                oa_v[buf, pl.ds(jb, _L)] = acc0 * rden

            d2.start()

    start_in(0, 0)

    def pair_body(p, _):
        i0 = 2 * p
        start_in(i0 + 1, 1)
        wait_in(i0, 0)

        @pl.when(i0 >= 2)
        def _():
            wait_out(i0 - 2, 0)

        compute(i0, 0)

        @pl.when(i0 + 2 < _CHUNKS)
        def _():
            start_in(i0 + 2, 0)

        wait_in(i0 + 1, 1)

        @pl.when(i0 >= 2)
        def _():
            wait_out(i0 - 1, 1)

        compute(i0 + 1, 1)
        return 0

    lax.fori_loop(0, _CHUNKS // 2, pair_body, 0)
    wait_out(_CHUNKS - 2, 0)
    wait_out(_CHUNKS - 1, 1)


@jax.jit
def _composite_sc(idxT, distsT, tab01, tab2):
    mesh = plsc.VectorSubcoreMesh(core_axis_name="c", subcore_axis_name="s")
    return pl.kernel(
        _sc_body,
        out_type=(jax.ShapeDtypeStruct((_N,), jnp.float32),
                  jax.ShapeDtypeStruct((_N,), jnp.float32),
                  jax.ShapeDtypeStruct((_N,), jnp.float32)),
        mesh=mesh,
        compiler_params=pltpu.CompilerParams(needs_layout_passes=False),
        scratch_types=[
            pltpu.VMEM((_P,), jnp.int32),
            pltpu.VMEM((_CHUNK * _K,), jnp.int32),
            pltpu.VMEM((_CHUNK * _K,), jnp.int32),
            pltpu.VMEM((_CHUNK * _K,), jnp.float32),
            pltpu.VMEM((_CHUNK * _K,), jnp.float32),
            pltpu.VMEM((2, _CHUNK), jnp.float32),
            pltpu.VMEM((2, _CHUNK), jnp.float32),
            pltpu.SemaphoreType.DMA((2,)),
            pltpu.SemaphoreType.DMA((2,)),
        ],
    )(idxT, distsT, tab01, tab2)


def kernel(idx, zbuf, dists, features):
    idxT = idx.reshape(_N * _K)
    distsT = dists.reshape(_N * _K)
    b0 = lax.bitcast_convert_type(
        features[:, 0].astype(jnp.bfloat16), jnp.uint16).astype(jnp.uint32)
    b1 = lax.bitcast_convert_type(
        features[:, 1].astype(jnp.bfloat16), jnp.uint16).astype(jnp.uint32)
    tab01 = lax.bitcast_convert_type((b0 << 16) | b1, jnp.int32)
    tab2 = lax.bitcast_convert_type(features[:, 2], jnp.int32)
    p0, p1, p2 = _composite_sc(idxT, distsT, tab01, tab2)
    rgb = jnp.stack([p0, p1, p2], axis=-1).reshape(_B, _H, _W, 3)
    return rgb, zbuf, idx


# speedup vs baseline: 2.6982x; 2.6982x over previous
import jax
import jax.numpy as jnp
from jax import lax
from jax.experimental import pallas as pl
from jax.experimental.pallas import tpu as pltpu
from jax.experimental.pallas import tpu_sc as plsc

_RADIUS = 0.01
_B, _H, _W, _K = 4, 512, 512, 16
_P = 100000
_N = _B * _H * _W

_NC, _NS, _L = 2, 16, 16
_NW = _NC * _NS
_GROUPS = _NW // 2
_PX_PER_GROUP = _N // _GROUPS
_CHUNK = 256
_CHUNKS = _PX_PER_GROUP // _CHUNK


def _sc_body(idxT, distsT, tab01, tab2, out0, out1, out2,
             tab_v, idx_v0, idx_v1, dst_v0, dst_v1, oa_v, ob_v,
             sem_in, sem_out):
    idx_bufs = (idx_v0, idx_v1)
    dst_bufs = (dst_v0, dst_v1)
    wid = lax.axis_index("s") * _NC + lax.axis_index("c")
    role = wid % 2
    group = wid // 2
    gbase = group * _PX_PER_GROUP
    inv_r2 = 1.0 / (_RADIUS * _RADIUS)

    @pl.when(role == 0)
    def _():
        pltpu.sync_copy(tab01, tab_v)

    @pl.when(role == 1)
    def _():
        pltpu.sync_copy(tab2, tab_v)

    def in_descs(ci, buf):
        off = gbase + ci * _CHUNK
        return (
            pltpu.make_async_copy(
                idxT.at[:, pl.ds(off, _CHUNK)], idx_bufs[buf],
                sem_in.at[buf]),
            pltpu.make_async_copy(
                distsT.at[:, pl.ds(off, _CHUNK)], dst_bufs[buf],
                sem_in.at[buf]),
        )

    def start_in(ci, buf):
        for d in in_descs(ci, buf):
            d.start()

    def wait_in(ci, buf):
        for d in in_descs(ci, buf):
            d.wait()

    def out_descs(ci, buf):
        off = gbase + ci * _CHUNK
        return (
            pltpu.make_async_copy(
                oa_v.at[buf], out0.at[pl.ds(off, _CHUNK)], sem_out.at[buf]),
            pltpu.make_async_copy(
                ob_v.at[buf], out1.at[pl.ds(off, _CHUNK)], sem_out.at[buf]),
            pltpu.make_async_copy(
                oa_v.at[buf], out2.at[pl.ds(off, _CHUNK)], sem_out.at[buf]),
        )

    def wait_out(ci, buf):
        d0, d1, d2 = out_descs(ci, buf)

        @pl.when(role == 0)
        def _():
            d0.wait()
            d1.wait()

        @pl.when(role == 1)
        def _():
            d2.wait()

    def compute(ci, buf):
        off = gbase + ci * _CHUNK
        ib = idx_bufs[buf]
        db = dst_bufs[buf]
        d0, d1, d2 = out_descs(ci, buf)

        @pl.when(role == 0)
        def _():
            @plsc.parallel_loop(0, _CHUNK // _L)
            def _(j):
                jb = j * _L
                zero = jnp.zeros((_L,), jnp.float32)
                acc0, acc1, den = zero, zero, zero
                for k in range(_K):
                    iv = ib[k, pl.ds(jb, _L)]
                    dv = db[k, pl.ds(jb, _L)]
                    w = 1.0 - dv * inv_r2
                    word = plsc.load_gather(tab_v, [iv])
                    c0 = plsc.bitcast(word & jnp.int32(-65536), jnp.float32)
                    c1 = plsc.bitcast(lax.shift_left(word, 16), jnp.float32)
                    acc0 = acc0 + w * c0
                    acc1 = acc1 + w * c1
                    den = den + w
                rden = 1.0 / jnp.maximum(den, 1e-10)
                oa_v[buf, pl.ds(jb, _L)] = acc0 * rden
                ob_v[buf, pl.ds(jb, _L)] = acc1 * rden

            d0.start()
            d1.start()

        @pl.when(role == 1)
        def _():
            @plsc.parallel_loop(0, _CHUNK // _L)
            def _(j):
                jb = j * _L
                zero = jnp.zeros((_L,), jnp.float32)
                acc0, den = zero, zero
                for k in range(_K):
                    iv = ib[k, pl.ds(jb, _L)]
                    dv = db[k, pl.ds(jb, _L)]
                    w = 1.0 - dv * inv_r2
                    word = plsc.load_gather(tab_v, [iv])
                    acc0 = acc0 + w * plsc.bitcast(word, jnp.float32)
                    den = den + w
                rden = 1.0 / jnp.maximum(den, 1e-10)
                oa_v[buf, pl.ds(jb, _L)] = acc0 * rden

            d2.start()

    start_in(0, 0)

    def pair_body(p, _):
        i0 = 2 * p
        start_in(i0 + 1, 1)
        wait_in(i0, 0)

        @pl.when(i0 >= 2)
        def _():
            wait_out(i0 - 2, 0)

        compute(i0, 0)

        @pl.when(i0 + 2 < _CHUNKS)
        def _():
            start_in(i0 + 2, 0)

        wait_in(i0 + 1, 1)

        @pl.when(i0 >= 2)
        def _():
            wait_out(i0 - 1, 1)

        compute(i0 + 1, 1)
        return 0

    lax.fori_loop(0, _CHUNKS // 2, pair_body, 0)
    wait_out(_CHUNKS - 2, 0)
    wait_out(_CHUNKS - 1, 1)


@jax.jit
def _composite_sc(idxT, distsT, tab01, tab2):
    mesh = plsc.VectorSubcoreMesh(core_axis_name="c", subcore_axis_name="s")
    return pl.kernel(
        _sc_body,
        out_type=(jax.ShapeDtypeStruct((_N,), jnp.float32),
                  jax.ShapeDtypeStruct((_N,), jnp.float32),
                  jax.ShapeDtypeStruct((_N,), jnp.float32)),
        mesh=mesh,
        compiler_params=pltpu.CompilerParams(needs_layout_passes=False),
        scratch_types=[
            pltpu.VMEM((_P,), jnp.int32),
            pltpu.VMEM((_K, _CHUNK), jnp.int32),
            pltpu.VMEM((_K, _CHUNK), jnp.int32),
            pltpu.VMEM((_K, _CHUNK), jnp.float32),
            pltpu.VMEM((_K, _CHUNK), jnp.float32),
            pltpu.VMEM((2, _CHUNK), jnp.float32),
            pltpu.VMEM((2, _CHUNK), jnp.float32),
            pltpu.SemaphoreType.DMA((2,)),
            pltpu.SemaphoreType.DMA((2,)),
        ],
    )(idxT, distsT, tab01, tab2)


def kernel(idx, zbuf, dists, features):
    idxT = idx.reshape(_N, _K).T
    distsT = dists.reshape(_N, _K).T
    b0 = lax.bitcast_convert_type(
        features[:, 0].astype(jnp.bfloat16), jnp.uint16).astype(jnp.uint32)
    b1 = lax.bitcast_convert_type(
        features[:, 1].astype(jnp.bfloat16), jnp.uint16).astype(jnp.uint32)
    tab01 = lax.bitcast_convert_type((b0 << 16) | b1, jnp.int32)
    tab2 = lax.bitcast_convert_type(features[:, 2], jnp.int32)
    p0, p1, p2 = _composite_sc(idxT, distsT, tab01, tab2)
    rgb = jnp.stack([p0, p1, p2], axis=-1).reshape(_B, _H, _W, 3)
    return rgb, zbuf, idx


# speedup vs baseline: 3.1811x; 1.1790x over previous
import jax
import jax.numpy as jnp
from jax import lax
from jax.experimental import pallas as pl
from jax.experimental.pallas import tpu as pltpu
from jax.experimental.pallas import tpu_sc as plsc

_RADIUS = 0.01
_B, _H, _W, _K = 4, 512, 512, 16
_P = 100000
_N = _B * _H * _W

_NC, _NS, _L = 2, 16, 16
_NW = _NC * _NS
_GROUPS = _NW // 2
_PX_PER_GROUP = _N // _GROUPS
_CHUNK = 512
_CHUNKS = _PX_PER_GROUP // _CHUNK

_IDX_MASK = jnp.int32((1 << 17) - 1)
_W_MASK = jnp.int32(-131072)


def _sc_body(packedT, tab01, tab2, out0, out1, out2,
             tab_v, pk_v0, pk_v1, oa_v, ob_v, sem_in, sem_out):
    pk_bufs = (pk_v0, pk_v1)
    wid = lax.axis_index("s") * _NC + lax.axis_index("c")
    role = wid % 2
    group = wid // 2
    gbase = group * _PX_PER_GROUP

    @pl.when(role == 0)
    def _():
        pltpu.sync_copy(tab01, tab_v)

    @pl.when(role == 1)
    def _():
        pltpu.sync_copy(tab2, tab_v)

    def in_desc(ci, buf):
        off = gbase + ci * _CHUNK
        return pltpu.make_async_copy(
            packedT.at[:, pl.ds(off, _CHUNK)], pk_bufs[buf], sem_in.at[buf])

    def out_descs(ci, buf):
        off = gbase + ci * _CHUNK
        return (
            pltpu.make_async_copy(
                oa_v.at[buf], out0.at[pl.ds(off, _CHUNK)], sem_out.at[buf]),
            pltpu.make_async_copy(
                ob_v.at[buf], out1.at[pl.ds(off, _CHUNK)], sem_out.at[buf]),
            pltpu.make_async_copy(
                oa_v.at[buf], out2.at[pl.ds(off, _CHUNK)], sem_out.at[buf]),
        )

    def wait_out(ci, buf):
        d0, d1, d2 = out_descs(ci, buf)

        @pl.when(role == 0)
        def _():
            d0.wait()
            d1.wait()

        @pl.when(role == 1)
        def _():
            d2.wait()

    def compute(ci, buf):
        ib = pk_bufs[buf]
        d0, d1, d2 = out_descs(ci, buf)

        @pl.when(role == 0)
        def _():
            @plsc.parallel_loop(0, _CHUNK // _L)
            def _(j):
                jb = j * _L
                zero = jnp.zeros((_L,), jnp.float32)
                acc0, acc1, den = zero, zero, zero
                for k in range(_K):
                    pk = ib[k, pl.ds(jb, _L)]
                    iv = pk & _IDX_MASK
                    w = plsc.bitcast(pk & _W_MASK, jnp.float32)
                    word = plsc.load_gather(tab_v, [iv])
                    c0 = plsc.bitcast(word, jnp.float32)
                    c1 = plsc.bitcast(lax.shift_left(word, 16), jnp.float32)
                    acc0 = acc0 + w * c0
                    acc1 = acc1 + w * c1
                    den = den + w
                rden = 1.0 / jnp.maximum(den, 1e-10)
                oa_v[buf, pl.ds(jb, _L)] = acc0 * rden
                ob_v[buf, pl.ds(jb, _L)] = acc1 * rden

            d0.start()
            d1.start()

        @pl.when(role == 1)
        def _():
            @plsc.parallel_loop(0, _CHUNK // _L)
            def _(j):
                jb = j * _L
                zero = jnp.zeros((_L,), jnp.float32)
                acc0, den = zero, zero
                for k in range(_K):
                    pk = ib[k, pl.ds(jb, _L)]
                    iv = pk & _IDX_MASK
                    w = plsc.bitcast(pk & _W_MASK, jnp.float32)
                    word = plsc.load_gather(tab_v, [iv])
                    acc0 = acc0 + w * plsc.bitcast(word, jnp.float32)
                    den = den + w
                rden = 1.0 / jnp.maximum(den, 1e-10)
                oa_v[buf, pl.ds(jb, _L)] = acc0 * rden

            d2.start()

    in_desc(0, 0).start()

    def pair_body(p, _):
        i0 = 2 * p
        in_desc(i0 + 1, 1).start()
        in_desc(i0, 0).wait()

        @pl.when(i0 >= 2)
        def _():
            wait_out(i0 - 2, 0)

        compute(i0, 0)

        @pl.when(i0 + 2 < _CHUNKS)
        def _():
            in_desc(i0 + 2, 0).start()

        in_desc(i0 + 1, 1).wait()

        @pl.when(i0 >= 2)
        def _():
            wait_out(i0 - 1, 1)

        compute(i0 + 1, 1)
        return 0

    lax.fori_loop(0, _CHUNKS // 2, pair_body, 0)
    wait_out(_CHUNKS - 2, 0)
    wait_out(_CHUNKS - 1, 1)


@jax.jit
def _composite_sc(packedT, tab01, tab2):
    mesh = plsc.VectorSubcoreMesh(core_axis_name="c", subcore_axis_name="s")
    return pl.kernel(
        _sc_body,
        out_type=(jax.ShapeDtypeStruct((_N,), jnp.float32),
                  jax.ShapeDtypeStruct((_N,), jnp.float32),
                  jax.ShapeDtypeStruct((_N,), jnp.float32)),
        mesh=mesh,
        compiler_params=pltpu.CompilerParams(needs_layout_passes=False),
        scratch_types=[
            pltpu.VMEM((_P,), jnp.int32),
            pltpu.VMEM((_K, _CHUNK), jnp.int32),
            pltpu.VMEM((_K, _CHUNK), jnp.int32),
            pltpu.VMEM((2, _CHUNK), jnp.float32),
            pltpu.VMEM((2, _CHUNK), jnp.float32),
            pltpu.SemaphoreType.DMA((2,)),
            pltpu.SemaphoreType.DMA((2,)),
        ],
    )(packedT, tab01, tab2)


def kernel(idx, zbuf, dists, features):
    w = 1.0 - dists / (_RADIUS * _RADIUS)
    wbits = lax.bitcast_convert_type(w, jnp.uint32)
    wtop = (wbits + jnp.uint32(0x10000)) & jnp.uint32(0xFFFE0000)
    packed = lax.bitcast_convert_type(
        wtop | idx.astype(jnp.uint32), jnp.int32)
    packedT = packed.reshape(_N, _K).T
    b0 = lax.bitcast_convert_type(
        features[:, 0].astype(jnp.bfloat16), jnp.uint16).astype(jnp.uint32)
    b1 = lax.bitcast_convert_type(
        features[:, 1].astype(jnp.bfloat16), jnp.uint16).astype(jnp.uint32)
    tab01 = lax.bitcast_convert_type((b0 << 16) | b1, jnp.int32)
    tab2 = lax.bitcast_convert_type(features[:, 2], jnp.int32)
    p0, p1, p2 = _composite_sc(packedT, tab01, tab2)
    rgb = jnp.stack([p0, p1, p2], axis=-1).reshape(_B, _H, _W, 3)
    return rgb, zbuf, idx
